# baseline (speedup 1.0000x reference)
import jax
import jax.numpy as jnp
from jax import lax
from jax.experimental import pallas as pl
from jax.experimental.pallas import tpu as pltpu
from jax.experimental.pallas import tpu_sc as plsc

N_CELL = 10000
N_NET = 10000
N_PIN = 320000
N_NN = 100000
L = 3

_NC, _NS = 2, 16
_NW = _NC * _NS
_CHUNK = 128
_NBUF = 4
_N_PAD = 10240
_ROWS_PER_TILE = _N_PAD // _NS

_LOG2 = 0.6931471805599453


def _ssp(x):
    return jax.nn.softplus(x) - _LOG2


def _mesh():
    return plsc.VectorSubcoreMesh(core_axis_name="c", subcore_axis_name="s")


_SC_PARAMS = pltpu.CompilerParams(use_tc_tiling_on_sc=False,
                                  needs_layout_passes=False)


def _zero_acc(zeros_hbm, buf, acc_sh, sid):
    pltpu.sync_copy(zeros_hbm, buf)
    row0 = sid * _ROWS_PER_TILE

    def zc(r, _):
        pltpu.sync_copy(buf, acc_sh.at[pl.ds(row0 + r * _CHUNK, _CHUNK)])
        return 0

    lax.fori_loop(0, _ROWS_PER_TILE // _CHUNK, zc, 0)


def _writeback(buf, acc_sh, out_ref, sid):
    row0 = sid * _ROWS_PER_TILE

    def wb(r, _):
        o = row0 + r * _CHUNK
        pltpu.sync_copy(acc_sh.at[pl.ds(o, _CHUNK)], buf)
        pltpu.sync_copy(buf, out_ref.at[pl.ds(o, _CHUNK)])
        return 0

    lax.fori_loop(0, _ROWS_PER_TILE // _CHUNK, wb, 0)


def _seg_plain_call(table, src2d, dst2d, C):
    G = C // _NBUF

    def body(table_hbm, src_hbm, dst_hbm, zeros_hbm, out_hbm,
             src_v, dst_v, rows_v, acc_sh, *sems):
        gsem = sems[:_NBUF]
        ssem = sems[_NBUF:]
        cid = lax.axis_index("c")
        sid = lax.axis_index("s")
        pltpu.sync_copy(src_hbm.at[cid, sid], src_v)
        pltpu.sync_copy(dst_hbm.at[cid, sid], dst_v)
        _zero_acc(zeros_hbm, rows_v.at[0], acc_sh, sid)
        plsc.subcore_barrier()

        for b in range(_NBUF):
            pltpu.async_copy(table_hbm.at[src_v.at[b]], rows_v.at[b], gsem[b])

        def group(g, _):
            for b in range(_NBUF):
                j = g * _NBUF + b
                pltpu.make_async_copy(table_hbm.at[src_v.at[j]],
                                      rows_v.at[b], gsem[b]).wait()
                pltpu.async_copy(rows_v.at[b], acc_sh.at[dst_v.at[j]],
                                 ssem[b], add=True)
            for b in range(_NBUF):
                j = g * _NBUF + b
                pltpu.make_async_copy(rows_v.at[b],
                                      acc_sh.at[dst_v.at[j]], ssem[b]).wait()
                pltpu.async_copy(table_hbm.at[src_v.at[j + _NBUF]],
                                 rows_v.at[b], gsem[b])
            return 0

        lax.fori_loop(0, G, group, 0)
        for b in range(_NBUF):
            pltpu.make_async_copy(table_hbm.at[src_v.at[b]],
                                  rows_v.at[b], gsem[b]).wait()
        plsc.subcore_barrier()
        _writeback(rows_v.at[0], acc_sh, out_hbm.at[cid], sid)

    k = pl.kernel(
        body,
        out_type=jax.ShapeDtypeStruct((_NC, _N_PAD, 64), jnp.float32),
        mesh=_mesh(),
        scratch_types=[
            pltpu.VMEM((C + _NBUF, _CHUNK), jnp.int32),
            pltpu.VMEM((C + _NBUF, _CHUNK), jnp.int32),
            pltpu.VMEM((_NBUF, _CHUNK, 64), jnp.float32),
            pltpu.VMEM_SHARED((_N_PAD, 64), jnp.float32),
        ] + [pltpu.SemaphoreType.DMA] * (2 * _NBUF),
        compiler_params=_SC_PARAMS,
    )
    zeros = jnp.zeros((_CHUNK, 64), jnp.float32)
    return k(table, src2d, dst2d, zeros)


def _cf_call(table, src2d, dst2d, he, C):
    NB = 2
    G = C // NB

    def body(table_hbm, src_hbm, dst_hbm, he_hbm, zeros_hbm, out_hbm,
             src_v, dst_v, rows_v, he_v, prod_v, acc_sh, *sems):
        gsem = sems[:NB]
        hsem = sems[NB:2 * NB]
        psem = sems[2 * NB:]
        cid = lax.axis_index("c")
        sid = lax.axis_index("s")
        pltpu.sync_copy(src_hbm.at[cid, sid], src_v)
        pltpu.sync_copy(dst_hbm.at[cid, sid], dst_v)
        _zero_acc(zeros_hbm, prod_v.at[0], acc_sh, sid)
        plsc.subcore_barrier()

        ebase = (cid * _NS + sid) * (C + _NBUF) * _CHUNK

        def he_slice(j):
            return he_hbm.at[pl.ds(ebase + j * _CHUNK, _CHUNK)]

        for b in range(NB):
            pltpu.async_copy(table_hbm.at[src_v.at[b]], rows_v.at[b], gsem[b])
            pltpu.async_copy(he_slice(b), he_v.at[b], hsem[b])

        def group(g, first):
            for b in range(NB):
                j = g * NB + b
                pb = b % 2
                pltpu.make_async_copy(table_hbm.at[src_v.at[j]],
                                      rows_v.at[b], gsem[b]).wait()
                pltpu.make_async_copy(he_slice(j), he_v.at[b], hsem[b]).wait()
                if not (first and b < 2):
                    pltpu.make_async_copy(prod_v.at[pb],
                                          acc_sh.at[dst_v.at[j]], psem[pb]).wait()

                def mul(i, _):
                    for q in range(4):
                        s = pl.ds(q * 16, 16)
                        prod_v[pb, i, s] = rows_v[b, i, s] * he_v[b, i, s]
                    return 0

                lax.fori_loop(0, _CHUNK, mul, 0)
                pltpu.async_copy(prod_v.at[pb], acc_sh.at[dst_v.at[j]],
                                 psem[pb], add=True)
                pltpu.async_copy(table_hbm.at[src_v.at[j + NB]],
                                 rows_v.at[b], gsem[b])
                pltpu.async_copy(he_slice(j + NB), he_v.at[b], hsem[b])
            return 0

        group(0, True)
        lax.fori_loop(1, G, lambda g, c: group(g, False), 0)
        for b in range(NB):
            pltpu.make_async_copy(table_hbm.at[src_v.at[b]],
                                  rows_v.at[b], gsem[b]).wait()
            pltpu.make_async_copy(he_slice(b), he_v.at[b], hsem[b]).wait()
        for pb in range(2):
            pltpu.make_async_copy(prod_v.at[pb],
                                  acc_sh.at[dst_v.at[pb]], psem[pb]).wait()
        plsc.subcore_barrier()
        _writeback(prod_v.at[0], acc_sh, out_hbm.at[cid], sid)

    k = pl.kernel(
        body,
        out_type=jax.ShapeDtypeStruct((_NC, _N_PAD, 64), jnp.float32),
        mesh=_mesh(),
        scratch_types=[
            pltpu.VMEM((C + _NBUF, _CHUNK), jnp.int32),
            pltpu.VMEM((C + _NBUF, _CHUNK), jnp.int32),
            pltpu.VMEM((NB, _CHUNK, 64), jnp.float32),
            pltpu.VMEM((NB, _CHUNK, 64), jnp.float32),
            pltpu.VMEM((2, _CHUNK, 64), jnp.float32),
            pltpu.VMEM_SHARED((_N_PAD, 64), jnp.float32),
        ] + [pltpu.SemaphoreType.DMA] * (2 * NB + 2),
        compiler_params=_SC_PARAMS,
    )
    zeros = jnp.zeros((_CHUNK, 64), jnp.float32)
    return k(table, src2d, dst2d, he, zeros)


def _faso_call(tab_fa, tab_so, fs2, fd2, C):
    NB = 2
    G = C // NB

    def body(fa_hbm, so_hbm, fs_hbm, fd_hbm, zeros_hbm, ofa_hbm, oso_hbm,
             fs_v, fd_v, rfa_v, rso_v, afa_sh, aso_sh, *sems):
        gfa = sems[:NB]
        gso = sems[NB:2 * NB]
        sfa = sems[2 * NB:3 * NB]
        sso = sems[3 * NB:]
        cid = lax.axis_index("c")
        sid = lax.axis_index("s")
        pltpu.sync_copy(fs_hbm.at[cid, sid], fs_v)
        pltpu.sync_copy(fd_hbm.at[cid, sid], fd_v)
        _zero_acc(zeros_hbm, rfa_v.at[0], afa_sh, sid)
        _zero_acc(zeros_hbm, rso_v.at[0], aso_sh, sid)
        plsc.subcore_barrier()

        for b in range(NB):
            pltpu.async_copy(fa_hbm.at[fs_v.at[b]], rfa_v.at[b], gfa[b])
            pltpu.async_copy(so_hbm.at[fd_v.at[b]], rso_v.at[b], gso[b])

        def group(g, _):
            for b in range(NB):
                j = g * NB + b
                pltpu.make_async_copy(fa_hbm.at[fs_v.at[j]],
                                      rfa_v.at[b], gfa[b]).wait()
                pltpu.async_copy(rfa_v.at[b], afa_sh.at[fd_v.at[j]],
                                 sfa[b], add=True)
                pltpu.make_async_copy(so_hbm.at[fd_v.at[j]],
                                      rso_v.at[b], gso[b]).wait()
                pltpu.async_copy(rso_v.at[b], aso_sh.at[fs_v.at[j]],
                                 sso[b], add=True)
            for b in range(NB):
                j = g * NB + b
                pltpu.make_async_copy(rfa_v.at[b], afa_sh.at[fd_v.at[j]],
                                      sfa[b]).wait()
                pltpu.async_copy(fa_hbm.at[fs_v.at[j + NB]], rfa_v.at[b], gfa[b])
                pltpu.make_async_copy(rso_v.at[b], aso_sh.at[fs_v.at[j]],
                                      sso[b]).wait()
                pltpu.async_copy(so_hbm.at[fd_v.at[j + NB]], rso_v.at[b], gso[b])
            return 0

        lax.fori_loop(0, G, group, 0)
        for b in range(NB):
            pltpu.make_async_copy(fa_hbm.at[fs_v.at[b]], rfa_v.at[b], gfa[b]).wait()
            pltpu.make_async_copy(so_hbm.at[fd_v.at[b]], rso_v.at[b], gso[b]).wait()
        plsc.subcore_barrier()
        _writeback(rfa_v.at[0], afa_sh, ofa_hbm.at[cid], sid)
        _writeback(rso_v.at[0], aso_sh, oso_hbm.at[cid], sid)

    k = pl.kernel(
        body,
        out_type=[jax.ShapeDtypeStruct((_NC, _N_PAD, 64), jnp.float32),
                  jax.ShapeDtypeStruct((_NC, _N_PAD, 64), jnp.float32)],
        mesh=_mesh(),
        scratch_types=[
            pltpu.VMEM((C + _NBUF, _CHUNK), jnp.int32),
            pltpu.VMEM((C + _NBUF, _CHUNK), jnp.int32),
            pltpu.VMEM((NB, _CHUNK, 64), jnp.float32),
            pltpu.VMEM((NB, _CHUNK, 64), jnp.float32),
            pltpu.VMEM_SHARED((_N_PAD, 64), jnp.float32),
            pltpu.VMEM_SHARED((_N_PAD, 64), jnp.float32),
        ] + [pltpu.SemaphoreType.DMA] * (4 * NB),
        compiler_params=_SC_PARAMS,
    )
    zeros = jnp.zeros((_CHUNK, 64), jnp.float32)
    return k(tab_fa, tab_so, fs2, fd2, zeros)


def _count2_call(d2a, d2b, C):
    def body(da_hbm, db_hbm, ones_hbm, zeros_hbm, oa_hbm, ob_hbm,
             da_v, db_v, ones_v, zbuf_v, acca_sh, accb_sh, sem):
        cid = lax.axis_index("c")
        sid = lax.axis_index("s")
        pltpu.sync_copy(da_hbm.at[cid, sid], da_v)
        pltpu.sync_copy(db_hbm.at[cid, sid], db_v)
        pltpu.sync_copy(ones_hbm, ones_v)
        pltpu.sync_copy(zeros_hbm, zbuf_v)
        row0 = sid * _ROWS_PER_TILE
        zrows = _ROWS_PER_TILE // 5

        def zc(r, _):
            pltpu.sync_copy(zbuf_v, acca_sh.at[pl.ds(row0 + r * zrows, zrows)])
            pltpu.sync_copy(zbuf_v, accb_sh.at[pl.ds(row0 + r * zrows, zrows)])
            return 0

        lax.fori_loop(0, _ROWS_PER_TILE // zrows, zc, 0)
        plsc.subcore_barrier()

        def fire(j, _):
            pltpu.async_copy(ones_v, acca_sh.at[da_v.at[j]], sem, add=True)
            pltpu.async_copy(ones_v, accb_sh.at[db_v.at[j]], sem, add=True)
            return 0

        lax.fori_loop(0, C, fire, 0)

        def drain(j, _):
            pltpu.make_async_copy(ones_v, acca_sh.at[da_v.at[j]], sem).wait()
            pltpu.make_async_copy(ones_v, accb_sh.at[db_v.at[j]], sem).wait()
            return 0

        lax.fori_loop(0, C, drain, 0)
        plsc.subcore_barrier()

        def wb(r, _):
            o = row0 + r * zrows
            pltpu.sync_copy(acca_sh.at[pl.ds(o, zrows)], zbuf_v)
            pltpu.sync_copy(zbuf_v, oa_hbm.at[cid, pl.ds(o, zrows)])
            pltpu.sync_copy(accb_sh.at[pl.ds(o, zrows)], zbuf_v)
            pltpu.sync_copy(zbuf_v, ob_hbm.at[cid, pl.ds(o, zrows)])
            return 0

        lax.fori_loop(0, _ROWS_PER_TILE // zrows, wb, 0)

    k = pl.kernel(
        body,
        out_type=[jax.ShapeDtypeStruct((_NC, _N_PAD, 8), jnp.float32),
                  jax.ShapeDtypeStruct((_NC, _N_PAD, 8), jnp.float32)],
        mesh=_mesh(),
        scratch_types=[
            pltpu.VMEM((d2a.shape[2], _CHUNK), jnp.int32),
            pltpu.VMEM((d2a.shape[2], _CHUNK), jnp.int32),
            pltpu.VMEM((_CHUNK, 8), jnp.float32),
            pltpu.VMEM((_ROWS_PER_TILE // 5, 8), jnp.float32),
            pltpu.VMEM_SHARED((_N_PAD, 8), jnp.float32),
            pltpu.VMEM_SHARED((_N_PAD, 8), jnp.float32),
            pltpu.SemaphoreType.DMA,
        ],
        compiler_params=_SC_PARAMS,
    )
    ones = jnp.ones((_CHUNK, 8), jnp.float32)
    zeros = jnp.zeros((_ROWS_PER_TILE // 5, 8), jnp.float32)
    return k(d2a, d2b, ones, zeros)


def _pair_readout_call(tab0_d, tab0_a, tab1_d, tab1_a, idx0_2d, idx1_2d, lin_d, lin_a):
    P = idx0_2d.shape[1]

    def body(t0d_h, t0a_h, t1d_h, t1a_h, i0_h, i1_h, ld_h, la_h,
             zd_h, za_h, t0d, t0a, t1d, t1a, i0, i1, ldv, lav, zdv, zav):
        cid = lax.axis_index("c")
        sid = lax.axis_index("s")
        wid = cid * _NS + sid
        pltpu.sync_copy(t0d_h, t0d)
        pltpu.sync_copy(t0a_h, t0a)
        pltpu.sync_copy(t1d_h, t1d)
        pltpu.sync_copy(t1a_h, t1a)
        pltpu.sync_copy(i0_h.at[wid], i0)
        pltpu.sync_copy(i1_h.at[wid], i1)
        pltpu.sync_copy(ld_h.at[wid], ldv)
        pltpu.sync_copy(la_h.at[wid], lav)

        def step(j, _):
            s = pl.ds(j * 16, 16)
            a0 = i0[s]
            a1 = i1[s]
            g0d = plsc.load_gather(t0d, [a0])
            g1d = plsc.load_gather(t1d, [a1])
            g0a = plsc.load_gather(t0a, [a0])
            g1a = plsc.load_gather(t1a, [a1])
            zdv[s] = g0d + g1d + ldv[s]
            zav[s] = g0a + g1a + lav[s]
            return 0

        lax.fori_loop(0, P // 16, step, 0)
        pltpu.sync_copy(zdv, zd_h.at[wid])
        pltpu.sync_copy(zav, za_h.at[wid])

    k = pl.kernel(
        body,
        out_type=[jax.ShapeDtypeStruct((_NW, P), jnp.float32),
                  jax.ShapeDtypeStruct((_NW, P), jnp.float32)],
        mesh=_mesh(),
        scratch_types=[
            pltpu.VMEM((_N_PAD,), jnp.float32),
            pltpu.VMEM((_N_PAD,), jnp.float32),
            pltpu.VMEM((_N_PAD,), jnp.float32),
            pltpu.VMEM((_N_PAD,), jnp.float32),
            pltpu.VMEM((P,), jnp.int32),
            pltpu.VMEM((P,), jnp.int32),
            pltpu.VMEM((P,), jnp.float32),
            pltpu.VMEM((P,), jnp.float32),
            pltpu.VMEM((P,), jnp.float32),
            pltpu.VMEM((P,), jnp.float32),
        ],
        compiler_params=_SC_PARAMS,
    )
    return k(tab0_d, tab0_a, tab1_d, tab1_a, idx0_2d, idx1_2d, lin_d, lin_a)


def _readout_act_kernel(zd_ref, za_ref, dis_ref, ang_ref):
    zd = zd_ref[...]
    za = za_ref[...]
    dis_ref[...] = jax.nn.softplus(zd)
    ang_ref[...] = 2.0 * jax.nn.sigmoid(za)


def _readout_act(zd, za):
    n = zd.shape[0]
    zd = zd.reshape(n // 1000, 1000)
    za = za.reshape(n // 1000, 1000)
    dis, ang = pl.pallas_call(
        _readout_act_kernel,
        out_shape=[jax.ShapeDtypeStruct(zd.shape, jnp.float32),
                   jax.ShapeDtypeStruct(zd.shape, jnp.float32)],
    )(zd, za)
    return dis.reshape(n), ang.reshape(n)


def _pad_table(t):
    return jnp.pad(t, ((0, _N_PAD - t.shape[0]), (0, 0)))


def _edge_split(idx, C, fill):
    e = idx.shape[0]
    e_pad = _NW * (C + _NBUF) * _CHUNK
    p = jnp.pad(idx.astype(jnp.int32), (0, e_pad - e), constant_values=fill)
    return p.reshape(_NC, _NS, C + _NBUF, _CHUNK)


def _chunks_for(e):
    per = -(-e // (_NW * _CHUNK))
    return -(-per // _NBUF) * _NBUF


def kernel(cell_feat, net_feat, pin_feat, pins_edge_index, net_net_edge_index, net_net_pair_matrix, net_cell_pair_matrix, cell_lin_W, cell_lin_b, net_lin_W, net_lin_b, pin_lin_W, pin_lin_b, pins_W, pins_b, fa_W, fa_b, so_W, so_b, cf_node_W, cf_node_b, cf_e1_W, cf_e1_b, cf_e2_W, cf_e2_b, cf_out_W, cf_out_b, net_dis_W, net_dis_b, net_angle_W, net_angle_b, pin_dis_W, pin_dis_b, pin_angle_W, pin_angle_b):
    hc = jnp.tanh(cell_feat @ cell_lin_W + cell_lin_b)
    hn = jnp.tanh(net_feat @ net_lin_W + net_lin_b)
    hp = jnp.tanh(pin_feat @ pin_lin_W + pin_lin_b)

    c_idx, n_idx = pins_edge_index[0], pins_edge_index[1]
    f_src, f_dst = net_net_edge_index[0], net_net_edge_index[1]

    C_PIN = _chunks_for(N_PIN)
    C_NN = _chunks_for(N_NN)
    c2 = _edge_split(c_idx, C_PIN, N_CELL)
    n2 = _edge_split(n_idx, C_PIN, N_NET)
    fs2 = _edge_split(f_src, C_NN, N_NET)
    fd2 = _edge_split(f_dst, C_NN, N_NET)

    def inv_sqrt(cnt, n):
        deg = cnt[0, :n, 0] + cnt[1, :n, 0]
        return jnp.where(deg > 0, deg, 1.0) ** -0.5

    cnt_c, cnt_n = _count2_call(c2, n2, C_PIN)
    cnt_fs, cnt_fd = _count2_call(fs2, fd2, C_NN)
    c_cell_out = inv_sqrt(cnt_c, N_CELL)
    c_net_in = inv_sqrt(cnt_n, N_NET)
    c_fsrc = inv_sqrt(cnt_fs, N_NET)
    c_fdst = inv_sqrt(cnt_fd, N_NET)

    e_full = _NW * (C_PIN + _NBUF) * _CHUNK
    hp_pad = jnp.pad(hp, ((0, e_full - N_PIN), (0, 0)))

    for l in range(L):
        h_pins = _pad_table((hc * c_cell_out[:, None]) @ pins_W[l])
        h_fa = _pad_table((hn * c_fsrc[:, None]) @ fa_W[l])
        h_so = _pad_table((hn * c_fdst[:, None]) @ so_W[l])

        agg = _seg_plain_call(h_pins, c2, n2, C_PIN)
        net_pins = (agg[0, :N_NET] + agg[1, :N_NET]) * c_net_in[:, None] + pins_b[l]
        afa, aso = _faso_call(h_fa, h_so, fs2, fd2, C_NN)
        net_fa = (afa[0, :N_NET] + afa[1, :N_NET]) * c_fdst[:, None] + fa_b[l]
        net_so = (aso[0, :N_NET] + aso[1, :N_NET]) * c_fsrc[:, None] + so_b[l]
        new_net = jnp.maximum(jnp.maximum(net_pins, net_fa), net_so)

        hv = _pad_table(hn @ cf_node_W[l] + cf_node_b[l])
        he = _ssp(_ssp(hp_pad @ cf_e1_W[l] + cf_e1_b[l]) @ cf_e2_W[l] + cf_e2_b[l])
        agg = _cf_call(hv, n2, c2, he, C_PIN)
        h = agg[0, :N_CELL] + agg[1, :N_CELL]
        new_cell = _ssp(h @ cf_out_W[l] + cf_out_b[l])

        hc, hn = new_cell, new_net

    p0 = net_net_pair_matrix[:, 0].astype(jnp.int32)
    p1 = net_net_pair_matrix[:, 1].astype(jnp.int32)
    q0 = net_cell_pair_matrix[:, 0].astype(jnp.int32)
    q1 = net_cell_pair_matrix[:, 1].astype(jnp.int32)

    hn_pad = _pad_table(hn)
    hc_pad = _pad_table(hc)

    w_nn = jnp.concatenate([net_dis_W[:64], net_angle_W[:64], net_dis_W[64:], net_angle_W[64:]], axis=1)
    s_nn = hn_pad @ w_nn
    t_nn_d0 = s_nn[:, 0] + net_dis_b[0]
    t_nn_a0 = s_nn[:, 1] + net_angle_b[0]

    ep_nn = -(-N_NN // (_NW * 16)) * (_NW * 16)
    pnn = ep_nn // _NW
    p0_2d = jnp.pad(p0, (0, ep_nn - N_NN)).reshape(_NW, pnn)
    p1_2d = jnp.pad(p1, (0, ep_nn - N_NN)).reshape(_NW, pnn)
    zeros_nn = jnp.zeros((_NW, pnn), jnp.float32)
    z_nd, z_na = _pair_readout_call(t_nn_d0, t_nn_a0, s_nn[:, 2], s_nn[:, 3],
                                    p0_2d, p1_2d, zeros_nn, zeros_nn)
    z_nd = z_nd.reshape(-1)[:N_NN]
    z_na = z_na.reshape(-1)[:N_NN]

    w_np = jnp.concatenate([pin_dis_W[:64], pin_angle_W[:64]], axis=1)
    w_cp = jnp.concatenate([pin_dis_W[80:], pin_angle_W[80:]], axis=1)
    w_pp = jnp.concatenate([pin_dis_W[64:80], pin_angle_W[64:80]], axis=1)
    s_np = hn_pad @ w_np
    s_cp = hc_pad @ w_cp
    s_pp = hp @ w_pp

    ep_pin = -(-N_PIN // (_NW * 16)) * (_NW * 16)
    ppin = ep_pin // _NW
    q0_2d = jnp.pad(q0, (0, ep_pin - N_PIN)).reshape(_NW, ppin)
    q1_2d = jnp.pad(q1, (0, ep_pin - N_PIN)).reshape(_NW, ppin)
    lin_d = jnp.pad(s_pp[:, 0] + pin_dis_b[0], (0, ep_pin - N_PIN)).reshape(_NW, ppin)
    lin_a = jnp.pad(s_pp[:, 1] + pin_angle_b[0], (0, ep_pin - N_PIN)).reshape(_NW, ppin)
    z_pd, z_pa = _pair_readout_call(s_np[:, 0], s_np[:, 1], s_cp[:, 0], s_cp[:, 1],
                                    q0_2d, q1_2d, lin_d, lin_a)
    z_pd = z_pd.reshape(-1)[:N_PIN]
    z_pa = z_pa.reshape(-1)[:N_PIN]

    net_dis, net_angle = _readout_act(z_nd, z_na)
    pin_dis, pin_angle = _readout_act(z_pd, z_pa)
    return (net_dis, net_angle, pin_dis, pin_angle)
